# baseline (speedup 1.0000x reference)
import jax
import jax.numpy as jnp
from jax.experimental import pallas as pl
from jax.experimental.pallas import tpu as pltpu

B, S, F = 32, 2048, 512
SEG = 256
NSEM = 8


def _bulk_tc(x, indices, starts):
    def body(idx_ref, st_ref, x_ref, out_ref, *sems):
        hs = []
        for i in range(B):
            h = pltpu.make_async_copy(x_ref.at[i], out_ref.at[i], sems[i % NSEM])
            h.start()
            hs.append(h)
        for h in hs:
            h.wait()

    return pl.pallas_call(
        body,
        out_shape=jax.ShapeDtypeStruct((B, S, F), jnp.float32),
        in_specs=[
            pl.BlockSpec(memory_space=pltpu.SMEM),
            pl.BlockSpec(memory_space=pltpu.SMEM),
            pl.BlockSpec(memory_space=pl.ANY),
        ],
        out_specs=pl.BlockSpec(memory_space=pl.ANY),
        scratch_shapes=[pltpu.SemaphoreType.DMA] * NSEM,
    )(indices, starts, x)


def kernel(x, indices, starts):
    return _bulk_tc(x, indices, starts)


# speedup vs baseline: 33.4111x; 33.4111x over previous
import functools

import jax
import jax.numpy as jnp
from jax import lax
from jax.experimental import pallas as pl
from jax.experimental.pallas import tpu as pltpu
from jax.experimental.pallas import tpu_sc as plsc

B, S, F = 32, 2048, 512
SEG = 256
LANES = 16
CH = 32
NCH = S // CH
NBUF = 6


def _cutmix_sc(x, indices, starts):
    mesh = plsc.VectorSubcoreMesh(core_axis_name="c", subcore_axis_name="s")
    info = plsc.get_sparse_core_info()
    nc = info.num_cores

    @functools.partial(
        pl.kernel,
        mesh=mesh,
        out_type=jax.ShapeDtypeStruct((B, S, F), jnp.float32),
        scratch_types=(
            [pltpu.VMEM((B + 16,), jnp.int32)] * 2
            + [pltpu.VMEM((CH, F), jnp.float32)] * NBUF
            + [pltpu.VMEM((8, F), jnp.float32)] * 2
            + [pltpu.SemaphoreType.DMA] * (2 * NBUF)
        ),
    )
    def k(x_hbm, idx_hbm, st_hbm, out_hbm, *scr):
        idx_v, st_v = scr[0], scr[1]
        bufs = scr[2:2 + NBUF]
        buf_i, buf_p = scr[2 + NBUF], scr[3 + NBUF]
        sin = scr[4 + NBUF:4 + 2 * NBUF]
        sout = scr[4 + 2 * NBUF:4 + 3 * NBUF]

        wid = lax.axis_index("s") * nc + lax.axis_index("c")
        pltpu.sync_copy(idx_hbm, idx_v.at[pl.ds(0, B)])
        pltpu.sync_copy(st_hbm, st_v.at[pl.ds(0, B)])
        p = idx_v[pl.ds(wid, LANES)][0]
        st = st_v[pl.ds(wid, LANES)][0]
        m = lax.rem(st, 8)
        q = lax.rem(st, CH)
        g = (q - m) // 8

        def start_in(c):
            b = c % NBUF
            c0 = c * CH
            inside = jnp.logical_and(st <= c0, c0 + CH <= st + SEG)
            src = lax.select(inside, p, wid)
            return pltpu.async_copy(
                x_hbm.at[src, pl.ds(c0, CH)], bufs[b], sin[b])

        def start_out(c):
            b = c % NBUF
            return pltpu.async_copy(
                bufs[b], out_hbm.at[wid, pl.ds(c * CH, CH)], sout[b])

        in_h = [None] * NBUF
        out_h = [None] * NBUF
        for c in range(NBUF - 1):
            in_h[c] = start_in(c)
        for c in range(NCH):
            b = c % NBUF
            if c + NBUF - 1 < NCH:
                b2 = (c + NBUF - 1) % NBUF
                if c >= 1:
                    out_h[b2].wait()
                in_h[b2] = start_in(c + NBUF - 1)
            in_h[b].wait()
            out_h[b] = start_out(c)
        for b in range(min(NBUF, NCH)):
            out_h[b].wait()

        def seg_copy(off, n):
            pltpu.sync_copy(
                x_hbm.at[p, pl.ds(off, n)], bufs[0].at[pl.ds(0, n)])
            pltpu.sync_copy(
                bufs[0].at[pl.ds(0, n)], out_hbm.at[wid, pl.ds(off, n)])

        def copy_8blocks(off, nblocks):
            for j in range(1, CH // 8):
                @pl.when(nblocks == j)
                def _arm():
                    seg_copy(pl.multiple_of(off, 8), 8 * j)

        def merge_edge(base, from_p):
            pltpu.sync_copy(x_hbm.at[wid, pl.ds(base, 8)], buf_i)
            pltpu.sync_copy(x_hbm.at[p, pl.ds(base, 8)], buf_p)
            for r in range(8):
                @pl.when(from_p(r))
                def _row():
                    for c in range(F // LANES):
                        sl = pl.ds(c * LANES, LANES)
                        buf_i[r, sl] = buf_p[r, sl]
            pltpu.sync_copy(buf_i, out_hbm.at[wid, pl.ds(base, 8)])

        @pl.when(jnp.logical_and(q != 0, m == 0))
        def _aligned8():
            copy_8blocks(st, (CH - q) // 8)
            copy_8blocks(st + SEG - q, g)

        @pl.when(m != 0)
        def _unaligned():
            a0 = pl.multiple_of(st - m, 8)
            b0 = pl.multiple_of(st + SEG - m, 8)
            copy_8blocks(a0 + 8, (CH - 8) // 8 - g)
            copy_8blocks(st + SEG - q, g)
            merge_edge(a0, lambda r: r >= m)
            merge_edge(b0, lambda r: r < m)

    return k(x, indices, starts)


def kernel(x, indices, starts):
    return _cutmix_sc(x, indices, starts)
